# baseline (speedup 1.0000x reference)
import functools

import jax
import jax.numpy as jnp
from jax.experimental import pallas as pl
from jax.experimental.pallas import tpu as pltpu
from jax.experimental.pallas import tpu_sc as plsc

_VOCAB = 256
_EMB = 128
_C = 256
_HID = 256
_NCLS = 20
_B = 16
_T = 1024
_BH0 = 10
_BH1 = _B - _BH0


def _sc_embed_gather(table, idx_flat):
    n = idx_flat.shape[0]
    nw = 32
    b_per_w = n // nw
    mesh = plsc.VectorSubcoreMesh(core_axis_name="c", subcore_axis_name="s")

    @functools.partial(
        pl.kernel,
        mesh=mesh,
        out_type=jax.ShapeDtypeStruct((n, table.shape[1]), table.dtype),
        scratch_types=[
            pltpu.VMEM((b_per_w,), jnp.int32),
            pltpu.VMEM((b_per_w, table.shape[1]), table.dtype),
            pltpu.SemaphoreType.DMA,
        ],
    )
    def gather_kernel(table_hbm, idx_hbm, out_hbm, idx_v, rows_v, sem):
        wid = jax.lax.axis_index("s") * 2 + jax.lax.axis_index("c")
        base = wid * b_per_w
        pltpu.sync_copy(idx_hbm.at[pl.ds(base, b_per_w)], idx_v)
        pltpu.async_copy(table_hbm.at[idx_v], rows_v, sem).wait()
        pltpu.sync_copy(rows_v, out_hbm.at[pl.ds(base, b_per_w)])

    return gather_kernel(table, idx_flat)


_RPS = 2


def _conv_step(emb_ref, w5_ref, w4_ref, w3_ref, b_ref, x_ref, write):
    n = _RPS * _T
    e = (emb_ref[...].reshape(n, _EMB) * 8.0).astype(jnp.float8_e4m3fn)
    x_ref[:, 0:_EMB] = e
    for dt in range(1, 5):
        x_ref[0 : n - dt, dt * _EMB : (dt + 1) * _EMB] = e[dt:n]
        x_ref[n - dt : n, dt * _EMB : (dt + 1) * _EMB] = jnp.zeros(
            (dt, _EMB), jnp.float8_e4m3fn
        )
    x = x_ref[...]

    for w_ref, width, slot in ((w5_ref, 5, 0), (w4_ref, 4, 1), (w3_ref, 3, 2)):
        length = _T - width + 1
        acc = jnp.dot(
            x[:, 0 : width * _EMB],
            w_ref[...],
            preferred_element_type=jnp.float32,
        )
        bias = b_ref[0, slot * _C : (slot + 1) * _C]
        for r in range(_RPS):
            m = jnp.max(acc[r * _T : r * _T + length], axis=0)
            write(r, slot, jax.nn.relu(m * (1.0 / 64.0) + bias))


def _conv_body(emb_ref, w5_ref, w4_ref, w3_ref, b_ref, out_ref, x_ref):
    def write(r, slot, f):
        out_ref[r, slot, :] = f

    _conv_step(emb_ref, w5_ref, w4_ref, w3_ref, b_ref, x_ref, write)


def _conv_feats(emb_bt, w5f, w4f, w3f, biases):
    return pl.pallas_call(
        _conv_body,
        grid=(_BH0 // _RPS,),
        in_specs=[
            pl.BlockSpec((_RPS, _T, _EMB), lambda b: (b, 0, 0)),
            pl.BlockSpec((5 * _EMB, _C), lambda b: (0, 0)),
            pl.BlockSpec((4 * _EMB, _C), lambda b: (0, 0)),
            pl.BlockSpec((3 * _EMB, _C), lambda b: (0, 0)),
            pl.BlockSpec((1, 3 * _C), lambda b: (0, 0)),
        ],
        out_specs=pl.BlockSpec((_RPS, 3, _C), lambda b: (b, 0, 0)),
        out_shape=jax.ShapeDtypeStruct((_BH0, 3, _C), jnp.float32),
        scratch_shapes=[pltpu.VMEM((_RPS * _T, 5 * _EMB), jnp.float8_e4m3fn)],
    )(emb_bt, w5f, w4f, w3f, biases)


def _gates_split(g):
    return (
        g[:, 0 * _HID : 1 * _HID],
        g[:, 1 * _HID : 2 * _HID],
        g[:, 2 * _HID : 3 * _HID],
        g[:, 3 * _HID : 4 * _HID],
    )


def _conv_rec_body(emb_ref, w5_ref, w4_ref, w3_ref, b_ref, feats0_ref,
                   wif_ref, whf_ref, bf_ref, wib_ref, bb_ref, pw_ref, pb_ref,
                   out_ref, x_ref, feats_ref):
    b = pl.program_id(0)

    def write(r, slot, f):
        feats_ref[b * _RPS + r, slot, :] = f

    _conv_step(emb_ref, w5_ref, w4_ref, w3_ref, b_ref, x_ref, write)

    @pl.when(b == _BH1 // _RPS - 1)
    def _recurrent():
        def seq(t):
            return jnp.concatenate(
                [feats0_ref[:, t, :], feats_ref[:, t, :]], axis=0
            )

        h = jnp.zeros((_B, _HID), jnp.float32)
        c = jnp.zeros((_B, _HID), jnp.float32)
        for t in range(3):
            x = seq(t).astype(jnp.bfloat16)
            g = (
                jnp.dot(x, wif_ref[...], preferred_element_type=jnp.float32)
                + jnp.dot(
                    h.astype(jnp.bfloat16),
                    whf_ref[...],
                    preferred_element_type=jnp.float32,
                )
                + bf_ref[0][None, :]
            )
            i, f, gg, o = _gates_split(g)
            c = jax.nn.sigmoid(f) * c + jax.nn.sigmoid(i) * jnp.tanh(gg)
            h = jax.nn.sigmoid(o) * jnp.tanh(c)

        xb = seq(2).astype(jnp.bfloat16)
        gb = jnp.dot(xb, wib_ref[...], preferred_element_type=jnp.float32)
        gb = gb + bb_ref[0][None, :]
        ib, fb, ggb, ob = _gates_split(gb)
        cb = jax.nn.sigmoid(ib) * jnp.tanh(ggb)
        hb = jax.nn.sigmoid(ob) * jnp.tanh(cb)

        last = jnp.concatenate([h, hb], axis=1).astype(jnp.bfloat16)
        logits = jnp.dot(last, pw_ref[...], preferred_element_type=jnp.float32)
        logits = logits + pb_ref[0][None, :]
        m = jnp.max(logits, axis=1, keepdims=True)
        s = logits - m
        out_ref[...] = s - jnp.log(jnp.sum(jnp.exp(s), axis=1, keepdims=True))


def _conv_rec(emb_bt, w5f, w4f, w3f, biases, feats0,
              wif_t, whf_t, bf, wib_t, bb, pw_t, pb):
    const = lambda b: (0, 0)
    return pl.pallas_call(
        _conv_rec_body,
        grid=(_BH1 // _RPS,),
        in_specs=[
            pl.BlockSpec((_RPS, _T, _EMB), lambda b: (b, 0, 0)),
            pl.BlockSpec((5 * _EMB, _C), const),
            pl.BlockSpec((4 * _EMB, _C), const),
            pl.BlockSpec((3 * _EMB, _C), const),
            pl.BlockSpec((1, 3 * _C), const),
            pl.BlockSpec((_BH0, 3, _C), lambda b: (0, 0, 0)),
            pl.BlockSpec((_C, 4 * _HID), const),
            pl.BlockSpec((_HID, 4 * _HID), const),
            pl.BlockSpec((1, 4 * _HID), const),
            pl.BlockSpec((_C, 4 * _HID), const),
            pl.BlockSpec((1, 4 * _HID), const),
            pl.BlockSpec((2 * _HID, _NCLS), const),
            pl.BlockSpec((1, _NCLS), const),
        ],
        out_specs=pl.BlockSpec((_B, _NCLS), const),
        out_shape=jax.ShapeDtypeStruct((_B, _NCLS), jnp.float32),
        scratch_shapes=[
            pltpu.VMEM((_RPS * _T, 5 * _EMB), jnp.float8_e4m3fn),
            pltpu.VMEM((_BH1, 3, _C), jnp.float32),
        ],
    )(emb_bt, w5f, w4f, w3f, biases, feats0,
      wif_t, whf_t, bf, wib_t, bb, pw_t, pb)


def kernel(inp, emb_table, conv_w5, conv_b5, conv_w4, conv_b4, conv_w3, conv_b3,
           W_ih_f, W_hh_f, b_ih_f, b_hh_f, W_ih_b, W_hh_b, b_ih_b, b_hh_b,
           proj_W, proj_b):
    idx = inp.T.reshape(-1)
    emb0 = _sc_embed_gather(emb_table, idx[: _BH0 * _T]).reshape(_BH0, _T, _EMB)
    emb1 = _sc_embed_gather(emb_table, idx[_BH0 * _T :]).reshape(_BH1, _T, _EMB)

    def flat_w(w, width):
        f = w[:, 0].transpose(2, 1, 0).reshape(width * _EMB, _C)
        return (f * 8.0).astype(jnp.float8_e4m3fn)

    w5f, w4f, w3f = flat_w(conv_w5, 5), flat_w(conv_w4, 4), flat_w(conv_w3, 3)
    biases = jnp.concatenate([conv_b5, conv_b4, conv_b3]).reshape(1, 3 * _C)

    feats0 = _conv_feats(emb0, w5f, w4f, w3f, biases)

    bf16 = jnp.bfloat16
    return _conv_rec(
        emb1, w5f, w4f, w3f, biases, feats0,
        W_ih_f.T.astype(bf16), W_hh_f.T.astype(bf16),
        (b_ih_f + b_hh_f).reshape(1, -1),
        W_ih_b.T.astype(bf16), (b_ih_b + b_hh_b).reshape(1, -1),
        proj_W.T.astype(bf16), proj_b.reshape(1, -1),
    )


# speedup vs baseline: 1.0076x; 1.0076x over previous
import functools

import jax
import jax.numpy as jnp
from jax.experimental import pallas as pl
from jax.experimental.pallas import tpu as pltpu
from jax.experimental.pallas import tpu_sc as plsc

_VOCAB = 256
_EMB = 128
_C = 256
_HID = 256
_NCLS = 20
_B = 16
_T = 1024
_BH0 = 12
_BH1 = _B - _BH0


def _sc_embed_gather(table, idx_flat):
    n = idx_flat.shape[0]
    nw = 32
    b_per_w = n // nw
    mesh = plsc.VectorSubcoreMesh(core_axis_name="c", subcore_axis_name="s")

    @functools.partial(
        pl.kernel,
        mesh=mesh,
        out_type=jax.ShapeDtypeStruct((n, table.shape[1]), table.dtype),
        scratch_types=[
            pltpu.VMEM((b_per_w,), jnp.int32),
            pltpu.VMEM((b_per_w, table.shape[1]), table.dtype),
            pltpu.SemaphoreType.DMA,
        ],
    )
    def gather_kernel(table_hbm, idx_hbm, out_hbm, idx_v, rows_v, sem):
        wid = jax.lax.axis_index("s") * 2 + jax.lax.axis_index("c")
        base = wid * b_per_w
        pltpu.sync_copy(idx_hbm.at[pl.ds(base, b_per_w)], idx_v)
        pltpu.async_copy(table_hbm.at[idx_v], rows_v, sem).wait()
        pltpu.sync_copy(rows_v, out_hbm.at[pl.ds(base, b_per_w)])

    return gather_kernel(table, idx_flat)


_RPS = 4


def _conv_step(emb_ref, w5_ref, w4_ref, w3_ref, b_ref, x_ref, write):
    n = _RPS * _T
    e = (emb_ref[...].reshape(n, _EMB) * 8.0).astype(jnp.float8_e4m3fn)
    x_ref[:, 0:_EMB] = e
    for dt in range(1, 5):
        x_ref[0 : n - dt, dt * _EMB : (dt + 1) * _EMB] = e[dt:n]
        x_ref[n - dt : n, dt * _EMB : (dt + 1) * _EMB] = jnp.zeros(
            (dt, _EMB), jnp.float8_e4m3fn
        )
    x = x_ref[...]

    for w_ref, width, slot in ((w5_ref, 5, 0), (w4_ref, 4, 1), (w3_ref, 3, 2)):
        length = _T - width + 1
        acc = jnp.dot(
            x[:, 0 : width * _EMB],
            w_ref[...],
            preferred_element_type=jnp.float32,
        )
        bias = b_ref[0, slot * _C : (slot + 1) * _C]
        for r in range(_RPS):
            m = jnp.max(acc[r * _T : r * _T + length], axis=0)
            write(r, slot, jax.nn.relu(m * (1.0 / 64.0) + bias))


def _conv_body(emb_ref, w5_ref, w4_ref, w3_ref, b_ref, out_ref, x_ref):
    def write(r, slot, f):
        out_ref[r, slot, :] = f

    _conv_step(emb_ref, w5_ref, w4_ref, w3_ref, b_ref, x_ref, write)


def _conv_feats(emb_bt, w5f, w4f, w3f, biases):
    return pl.pallas_call(
        _conv_body,
        grid=(_BH0 // _RPS,),
        in_specs=[
            pl.BlockSpec((_RPS, _T, _EMB), lambda b: (b, 0, 0)),
            pl.BlockSpec((5 * _EMB, _C), lambda b: (0, 0)),
            pl.BlockSpec((4 * _EMB, _C), lambda b: (0, 0)),
            pl.BlockSpec((3 * _EMB, _C), lambda b: (0, 0)),
            pl.BlockSpec((1, 3 * _C), lambda b: (0, 0)),
        ],
        out_specs=pl.BlockSpec((_RPS, 3, _C), lambda b: (b, 0, 0)),
        out_shape=jax.ShapeDtypeStruct((_BH0, 3, _C), jnp.float32),
        scratch_shapes=[pltpu.VMEM((_RPS * _T, 5 * _EMB), jnp.float8_e4m3fn)],
    )(emb_bt, w5f, w4f, w3f, biases)


def _gates_split(g):
    return (
        g[:, 0 * _HID : 1 * _HID],
        g[:, 1 * _HID : 2 * _HID],
        g[:, 2 * _HID : 3 * _HID],
        g[:, 3 * _HID : 4 * _HID],
    )


def _conv_rec_body(emb_ref, w5_ref, w4_ref, w3_ref, b_ref, feats0_ref,
                   wif_ref, whf_ref, bf_ref, wib_ref, bb_ref, pw_ref, pb_ref,
                   out_ref, x_ref, feats_ref):
    b = pl.program_id(0)

    def write(r, slot, f):
        feats_ref[b * _RPS + r, slot, :] = f

    _conv_step(emb_ref, w5_ref, w4_ref, w3_ref, b_ref, x_ref, write)

    @pl.when(b == _BH1 // _RPS - 1)
    def _recurrent():
        def seq(t):
            return jnp.concatenate(
                [feats0_ref[:, t, :], feats_ref[:, t, :]], axis=0
            )

        h = jnp.zeros((_B, _HID), jnp.float32)
        c = jnp.zeros((_B, _HID), jnp.float32)
        for t in range(3):
            x = seq(t).astype(jnp.bfloat16)
            g = (
                jnp.dot(x, wif_ref[...], preferred_element_type=jnp.float32)
                + jnp.dot(
                    h.astype(jnp.bfloat16),
                    whf_ref[...],
                    preferred_element_type=jnp.float32,
                )
                + bf_ref[0][None, :]
            )
            i, f, gg, o = _gates_split(g)
            c = jax.nn.sigmoid(f) * c + jax.nn.sigmoid(i) * jnp.tanh(gg)
            h = jax.nn.sigmoid(o) * jnp.tanh(c)

        xb = seq(2).astype(jnp.bfloat16)
        gb = jnp.dot(xb, wib_ref[...], preferred_element_type=jnp.float32)
        gb = gb + bb_ref[0][None, :]
        ib, fb, ggb, ob = _gates_split(gb)
        cb = jax.nn.sigmoid(ib) * jnp.tanh(ggb)
        hb = jax.nn.sigmoid(ob) * jnp.tanh(cb)

        last = jnp.concatenate([h, hb], axis=1).astype(jnp.bfloat16)
        logits = jnp.dot(last, pw_ref[...], preferred_element_type=jnp.float32)
        logits = logits + pb_ref[0][None, :]
        m = jnp.max(logits, axis=1, keepdims=True)
        s = logits - m
        out_ref[...] = s - jnp.log(jnp.sum(jnp.exp(s), axis=1, keepdims=True))


def _conv_rec(emb_bt, w5f, w4f, w3f, biases, feats0,
              wif_t, whf_t, bf, wib_t, bb, pw_t, pb):
    const = lambda b: (0, 0)
    return pl.pallas_call(
        _conv_rec_body,
        grid=(_BH1 // _RPS,),
        in_specs=[
            pl.BlockSpec((_RPS, _T, _EMB), lambda b: (b, 0, 0)),
            pl.BlockSpec((5 * _EMB, _C), const),
            pl.BlockSpec((4 * _EMB, _C), const),
            pl.BlockSpec((3 * _EMB, _C), const),
            pl.BlockSpec((1, 3 * _C), const),
            pl.BlockSpec((_BH0, 3, _C), lambda b: (0, 0, 0)),
            pl.BlockSpec((_C, 4 * _HID), const),
            pl.BlockSpec((_HID, 4 * _HID), const),
            pl.BlockSpec((1, 4 * _HID), const),
            pl.BlockSpec((_C, 4 * _HID), const),
            pl.BlockSpec((1, 4 * _HID), const),
            pl.BlockSpec((2 * _HID, _NCLS), const),
            pl.BlockSpec((1, _NCLS), const),
        ],
        out_specs=pl.BlockSpec((_B, _NCLS), const),
        out_shape=jax.ShapeDtypeStruct((_B, _NCLS), jnp.float32),
        scratch_shapes=[
            pltpu.VMEM((_RPS * _T, 5 * _EMB), jnp.float8_e4m3fn),
            pltpu.VMEM((_BH1, 3, _C), jnp.float32),
        ],
    )(emb_bt, w5f, w4f, w3f, biases, feats0,
      wif_t, whf_t, bf, wib_t, bb, pw_t, pb)


def kernel(inp, emb_table, conv_w5, conv_b5, conv_w4, conv_b4, conv_w3, conv_b3,
           W_ih_f, W_hh_f, b_ih_f, b_hh_f, W_ih_b, W_hh_b, b_ih_b, b_hh_b,
           proj_W, proj_b):
    idx = inp.T.reshape(-1)
    emb0 = _sc_embed_gather(emb_table, idx[: _BH0 * _T]).reshape(_BH0, _T, _EMB)
    emb1 = _sc_embed_gather(emb_table, idx[_BH0 * _T :]).reshape(_BH1, _T, _EMB)

    def flat_w(w, width):
        f = w[:, 0].transpose(2, 1, 0).reshape(width * _EMB, _C)
        return (f * 8.0).astype(jnp.float8_e4m3fn)

    w5f, w4f, w3f = flat_w(conv_w5, 5), flat_w(conv_w4, 4), flat_w(conv_w3, 3)
    biases = jnp.concatenate([conv_b5, conv_b4, conv_b3]).reshape(1, 3 * _C)

    feats0 = _conv_feats(emb0, w5f, w4f, w3f, biases)

    bf16 = jnp.bfloat16
    return _conv_rec(
        emb1, w5f, w4f, w3f, biases, feats0,
        W_ih_f.T.astype(bf16), W_hh_f.T.astype(bf16),
        (b_ih_f + b_hh_f).reshape(1, -1),
        W_ih_b.T.astype(bf16), (b_ih_b + b_hh_b).reshape(1, -1),
        proj_W.T.astype(bf16), proj_b.reshape(1, -1),
    )
